# baseline (speedup 1.0000x reference)
import functools

import jax
import jax.numpy as jnp
from jax import lax
from jax.experimental import pallas as pl
from jax.experimental.pallas import tpu as pltpu
from jax.experimental.pallas import tpu_sc as plsc

BATCH = 16384
HIDDEN = 32
_PACK = 4
_SROWS = 1000000 // _PACK

_NC = 2
_NS = 16
_NW = _NC * _NS
_BPW = BATCH // _NW
_CH = 128
_NCH = _BPW // _CH
_L = 16


def _sc_body(uI, iI, jI, uR, iR, ibias,
             x_out, l2_out,
             idxu_v, idxi_v, idxj_v, rowu_v, rowi_v, rowj_v,
             ubuf, ibuf, jbuf, ib_v, jb_v, x_v, l2_v, sem):
    wid = lax.axis_index("s") * _NC + lax.axis_index("c")
    base = wid * _BPW
    lane = lax.iota(jnp.int32, _L)

    l2acc0 = jnp.zeros((_L,), jnp.float32)

    def do_chunk(c, l2acc):
        off = base + c * _CH
        pltpu.sync_copy(uI.at[pl.ds(off, _CH)], idxu_v)
        pltpu.sync_copy(iI.at[pl.ds(off, _CH)], idxi_v)
        pltpu.sync_copy(jI.at[pl.ds(off, _CH)], idxj_v)

        for g in range(_CH // _L):
            sl = pl.ds(g * _L, _L)
            rowu_v[sl] = lax.shift_right_logical(idxu_v[sl], 2)
            rowi_v[sl] = lax.shift_right_logical(idxi_v[sl], 2)
            rowj_v[sl] = lax.shift_right_logical(idxj_v[sl], 2)

        copies = [
            pltpu.async_copy(uR.at[rowu_v], ubuf, sem),
            pltpu.async_copy(iR.at[rowi_v], ibuf, sem),
            pltpu.async_copy(iR.at[rowj_v], jbuf, sem),
            pltpu.async_copy(ibias.at[idxi_v], ib_v, sem),
            pltpu.async_copy(ibias.at[idxj_v], jb_v, sem),
        ]
        for cp in copies:
            cp.wait()

        def group(g, l2a):
            sl = pl.ds(g * _L, _L)
            svec = lane + g * _L
            colu = lax.shift_left(idxu_v[sl] & 3, 5)
            coli = lax.shift_left(idxi_v[sl] & 3, 5)
            colj = lax.shift_left(idxj_v[sl] & 3, 5)
            acc = ib_v[sl] - jb_v[sl]
            for h in range(HIDDEN):
                au = plsc.load_gather(ubuf, [svec, colu + h])
                ai = plsc.load_gather(ibuf, [svec, coli + h])
                aj = plsc.load_gather(jbuf, [svec, colj + h])
                acc = acc + au * (ai - aj)
                l2a = l2a + (au * au + ai * ai + aj * aj)
            x_v[pl.ds(c * _CH + g * _L, _L)] = acc
            return l2a

        return lax.fori_loop(0, _CH // _L, group, l2acc)

    l2acc = lax.fori_loop(0, _NCH, do_chunk, l2acc0)
    l2_v[...] = l2acc

    pltpu.sync_copy(x_v, x_out.at[pl.ds(base, _BPW)])
    pltpu.sync_copy(l2_v, l2_out.at[pl.ds(wid * _L, _L)])


def _sc_call(uI, iI, jI, uR, iR, item_bias):
    mesh = plsc.VectorSubcoreMesh(core_axis_name="c", subcore_axis_name="s")
    f = functools.partial(
        pl.kernel,
        mesh=mesh,
        compiler_params=pltpu.CompilerParams(use_tc_tiling_on_sc=True,
                                             needs_layout_passes=False),
        out_type=(
            jax.ShapeDtypeStruct((BATCH,), jnp.float32),
            jax.ShapeDtypeStruct((_NW * _L,), jnp.float32),
        ),
        scratch_types=[
            pltpu.VMEM((_CH,), jnp.int32),
            pltpu.VMEM((_CH,), jnp.int32),
            pltpu.VMEM((_CH,), jnp.int32),
            pltpu.VMEM((_CH,), jnp.int32),
            pltpu.VMEM((_CH,), jnp.int32),
            pltpu.VMEM((_CH,), jnp.int32),
            pltpu.VMEM((_CH, 128), jnp.float32),
            pltpu.VMEM((_CH, 128), jnp.float32),
            pltpu.VMEM((_CH, 128), jnp.float32),
            pltpu.VMEM((_CH,), jnp.float32),
            pltpu.VMEM((_CH,), jnp.float32),
            pltpu.VMEM((_BPW,), jnp.float32),
            pltpu.VMEM((_L,), jnp.float32),
            pltpu.SemaphoreType.DMA,
        ],
    )(_sc_body)
    return f(uI, iI, jI, uR, iR, item_bias)


def _loss_body(x_ref, l2_ref, out_ref):
    x = x_ref[...]
    ls = jnp.minimum(x, 0.0) - jnp.log(1.0 + jnp.exp(-jnp.abs(x)))
    l2 = jnp.sum(l2_ref[...])
    out_ref[0, 0] = 0.0001 * l2 - jnp.mean(ls)


def _tc_loss(x, l2p):
    return pl.pallas_call(
        _loss_body,
        out_shape=jax.ShapeDtypeStruct((1, 1), jnp.float32),
        out_specs=pl.BlockSpec(memory_space=pltpu.SMEM),
    )(x, l2p)


def kernel(input, user_embedding, item_embedding, item_bias):
    idx = input.astype(jnp.int32)
    uR = user_embedding.reshape(_SROWS, _PACK * HIDDEN)
    iR = item_embedding.reshape(_SROWS, _PACK * HIDDEN)
    x, l2p = _sc_call(idx[:, 0], idx[:, 1], idx[:, 2], uR, iR, item_bias)
    return _tc_loss(x, l2p).reshape(())
